# baseline (speedup 1.0000x reference)
import functools

import jax
import jax.numpy as jnp
from jax import lax
from jax.experimental import pallas as pl
from jax.experimental.pallas import tpu as pltpu
from jax.experimental.pallas import tpu_sc as plsc

_D = 256
_S = 512
_TAB = 1016
_NW = 32
_RPW = _S // _NW

_mesh = plsc.VectorSubcoreMesh(core_axis_name="c", subcore_axis_name="s")


@functools.partial(
    pl.kernel,
    out_type=jax.ShapeDtypeStruct((_S, _S, _D), jnp.float32),
    mesh=_mesh,
    scratch_types=[
        pltpu.SemaphoreType.DMA,
    ],
)
def _rpe_sc(a_hbm, out_hbm, sem):
    c = lax.axis_index("c")
    s = lax.axis_index("s")
    wid = s * 2 + c

    copies = []
    for r in range(_RPW):
        g = wid * _RPW + r
        k = (7 - r) % 8
        start = (_S - 1) - g
        q8 = pl.multiple_of(start - k, 8)
        copies.append(
            pltpu.async_copy(a_hbm.at[k, pl.ds(q8, _S), :],
                             out_hbm.at[g], sem)
        )
    for cp in copies:
        cp.wait()


def kernel(x, rel_embeddings):
    flipped = rel_embeddings[::-1]
    shifted = jnp.stack([flipped[k:k + _TAB] for k in range(8)])
    rel_pos = _rpe_sc(shifted)
    return (x, rel_pos)


# speedup vs baseline: 37.5290x; 37.5290x over previous
import functools

import jax
import jax.numpy as jnp
from jax import lax
from jax.experimental import pallas as pl
from jax.experimental.pallas import tpu as pltpu
from jax.experimental.pallas import tpu_sc as plsc

_D = 256
_S = 512
_TAB = 1016
_NW = 32
_RPW = _S // _NW

_mesh = plsc.VectorSubcoreMesh(core_axis_name="c", subcore_axis_name="s")


@functools.partial(
    pl.kernel,
    out_type=jax.ShapeDtypeStruct((_S, _S, _D), jnp.float32),
    mesh=_mesh,
    scratch_types=[
        pltpu.VMEM_SHARED((8, _TAB, _D), jnp.float32),
        pltpu.SemaphoreType.DMA,
    ],
)
def _rpe_sc(a_hbm, out_hbm, tab_spmem, sem):
    c = lax.axis_index("c")
    s = lax.axis_index("s")
    wid = s * 2 + c

    @pl.when(s < 8)
    def _stage_lo():
        pltpu.sync_copy(a_hbm.at[s, pl.ds(0, 512), :],
                        tab_spmem.at[s, pl.ds(0, 512), :])

    @pl.when(s >= 8)
    def _stage_hi():
        pltpu.sync_copy(a_hbm.at[s - 8, pl.ds(512, _TAB - 512), :],
                        tab_spmem.at[s - 8, pl.ds(512, _TAB - 512), :])

    plsc.subcore_barrier()

    copies = []
    for r in range(_RPW):
        g = wid * _RPW + r
        k = (7 - r) % 8
        start = (_S - 1) - g
        q8 = pl.multiple_of(start - k, 8)
        copies.append(
            pltpu.async_copy(tab_spmem.at[k, pl.ds(q8, _S), :],
                             out_hbm.at[g], sem)
        )
    for cp in copies:
        cp.wait()


def kernel(x, rel_embeddings):
    flipped = rel_embeddings[::-1]
    shifted = jnp.stack([flipped[k:k + _TAB] for k in range(8)])
    rel_pos = _rpe_sc(shifted)
    return (x, rel_pos)
